# baseline (device time: 26902 ns/iter reference)
import jax
import jax.numpy as jnp
from jax import lax
from jax.experimental import pallas as pl
from jax.experimental.pallas import tpu as pltpu

N_DEV = 32
HALVES = 2


def kernel(t):
    m, n = t.shape
    chunk = m // N_DEV
    half = chunk // HALVES

    def body(x_ref, out_ref, gather_ref, send1, recv1, send2, recv2):
        me = lax.axis_index("i")

        barrier = pltpu.get_barrier_semaphore()
        for k in range(1, N_DEV):
            pl.semaphore_signal(
                barrier, inc=1,
                device_id=((me + k) % N_DEV,),
                device_id_type=pl.DeviceIdType.MESH,
            )
        pl.semaphore_wait(barrier, N_DEV - 1)

        p1 = [[], []]
        for h in range(HALVES):
            for k in range(1, N_DEV):
                d = (me + k) % N_DEV
                rdma = pltpu.make_async_remote_copy(
                    src_ref=x_ref.at[pl.ds(d * chunk + h * half, half), :],
                    dst_ref=gather_ref.at[k, pl.ds(h * half, half), :],
                    send_sem=send1.at[h, k],
                    recv_sem=recv1.at[h, k],
                    device_id=(d,),
                    device_id_type=pl.DeviceIdType.MESH,
                )
                rdma.start()
                p1[h].append(rdma)

        gather_ref[0, :, :] = x_ref[pl.ds(me * chunk, chunk), :]

        p2 = [[], []]
        for h in range(HALVES):
            for rdma in p1[h]:
                rdma.wait_recv()
            s = jnp.sum(gather_ref[:, h * half:(h + 1) * half, :], axis=0)
            r = jnp.maximum(s, 0.0)
            y = jnp.tanh(s) * s * s + r * r * r
            out_ref[pl.ds(me * chunk + h * half, half), :] = y
            for k in range(1, N_DEV):
                d = (me + k) % N_DEV
                rdma = pltpu.make_async_remote_copy(
                    src_ref=out_ref.at[pl.ds(me * chunk + h * half, half), :],
                    dst_ref=out_ref.at[pl.ds(me * chunk + h * half, half), :],
                    send_sem=send2.at[h, k],
                    recv_sem=recv2.at[h, k],
                    device_id=(d,),
                    device_id_type=pl.DeviceIdType.MESH,
                )
                rdma.start()
                p2[h].append(rdma)

        for h in range(HALVES):
            for rdma in p2[h]:
                rdma.wait_recv()
            for rdma in p1[h]:
                rdma.wait_send()
            for rdma in p2[h]:
                rdma.wait_send()

    return pl.pallas_call(
        body,
        out_shape=jax.ShapeDtypeStruct((m, n), jnp.float32),
        in_specs=[pl.BlockSpec(memory_space=pltpu.VMEM)],
        out_specs=pl.BlockSpec(memory_space=pltpu.VMEM),
        scratch_shapes=[
            pltpu.VMEM((N_DEV, chunk, n), jnp.float32),
            pltpu.SemaphoreType.DMA((HALVES, N_DEV)),
            pltpu.SemaphoreType.DMA((HALVES, N_DEV)),
            pltpu.SemaphoreType.DMA((HALVES, N_DEV)),
            pltpu.SemaphoreType.DMA((HALVES, N_DEV)),
        ],
        compiler_params=pltpu.CompilerParams(collective_id=0),
    )(t)


# device time: 24890 ns/iter; 1.0808x vs baseline; 1.0808x over previous
import jax
import jax.numpy as jnp
from jax import lax
from jax.experimental import pallas as pl
from jax.experimental.pallas import tpu as pltpu

N_DEV = 32


def kernel(t):
    m, n = t.shape
    chunk = m // N_DEV

    def body(x_ref, out_ref, gather_ref, entry_sems, send1, recv1, send2, recv2):
        me = lax.axis_index("i")

        for k in range(1, N_DEV):
            pl.semaphore_signal(
                entry_sems.at[N_DEV - k], inc=1,
                device_id=((me + k) % N_DEV,),
                device_id_type=pl.DeviceIdType.MESH,
            )

        barrier = pltpu.get_barrier_semaphore()
        for nbr in ((me + 1) % N_DEV, (me - 1) % N_DEV):
            pl.semaphore_signal(
                barrier, inc=1,
                device_id=(nbr,),
                device_id_type=pl.DeviceIdType.MESH,
            )
        pl.semaphore_wait(barrier, 2)

        p1 = []
        for k in range(1, N_DEV):
            d = (me + k) % N_DEV
            pl.semaphore_wait(entry_sems.at[k], 1)
            rdma = pltpu.make_async_remote_copy(
                src_ref=x_ref.at[pl.ds(d * chunk, chunk), :],
                dst_ref=gather_ref.at[k],
                send_sem=send1.at[k],
                recv_sem=recv1.at[k],
                device_id=(d,),
                device_id_type=pl.DeviceIdType.MESH,
            )
            rdma.start()
            p1.append(rdma)

        gather_ref[0, :, :] = x_ref[pl.ds(me * chunk, chunk), :]

        for rdma in p1:
            rdma.wait_recv()

        s = jnp.sum(gather_ref[...], axis=0)
        r = jnp.maximum(s, 0.0)
        y = jnp.tanh(s) * s * s + r * r * r
        out_ref[pl.ds(me * chunk, chunk), :] = y

        p2 = []
        for k in range(1, N_DEV):
            d = (me + k) % N_DEV
            rdma = pltpu.make_async_remote_copy(
                src_ref=out_ref.at[pl.ds(me * chunk, chunk), :],
                dst_ref=out_ref.at[pl.ds(me * chunk, chunk), :],
                send_sem=send2.at[k],
                recv_sem=recv2.at[k],
                device_id=(d,),
                device_id_type=pl.DeviceIdType.MESH,
            )
            rdma.start()
            p2.append(rdma)

        for rdma in p1:
            rdma.wait_send()
        for rdma in p2:
            rdma.wait_recv()
        for rdma in p2:
            rdma.wait_send()

    return pl.pallas_call(
        body,
        out_shape=jax.ShapeDtypeStruct((m, n), jnp.float32),
        in_specs=[pl.BlockSpec(memory_space=pltpu.VMEM)],
        out_specs=pl.BlockSpec(memory_space=pltpu.VMEM),
        scratch_shapes=[
            pltpu.VMEM((N_DEV, chunk, n), jnp.float32),
            pltpu.SemaphoreType.REGULAR((N_DEV,)),
            pltpu.SemaphoreType.DMA((N_DEV,)),
            pltpu.SemaphoreType.DMA((N_DEV,)),
            pltpu.SemaphoreType.DMA((N_DEV,)),
            pltpu.SemaphoreType.DMA((N_DEV,)),
        ],
        compiler_params=pltpu.CompilerParams(collective_id=0),
    )(t)
